# Initial kernel scaffold; baseline (speedup 1.0000x reference)
import jax
import jax.numpy as jnp
from jax.experimental import pallas as pl


def kernel(X, edge_index, edge_weight, attention, conv_z_W, conv_z_b, lin_z_W, lin_z_b, conv_r_W, conv_r_b, lin_r_W, lin_r_b, conv_h_W, conv_h_b, lin_h_W, lin_h_b):
    raise NotImplementedError("write your pallas kernel here")


# speedup vs baseline: 150.1515x; 150.1515x over previous
import functools

import jax
import jax.numpy as jnp
from jax import lax
from jax.experimental import pallas as pl
from jax.experimental.pallas import tpu as pltpu
from jax.experimental.pallas import tpu_sc as plsc

N = 10000
E = 160000
PERIODS = 12
OUT = 64
W = 32
NP = 10240
EP = 163840
NC = 2
NS = 16
ROWS_PER_TILE = NP // (NC * NS)
ROWS_PER_SUB = NP // NS
CH1 = EP // NS // 128
CH3 = EP // (NC * NS) // 128
BN = 2048

@functools.cache
def _mesh():
    return plsc.VectorSubcoreMesh(core_axis_name="c", subcore_axis_name="s",
                                  num_cores=NC, num_subcores=NS)


def _k1_body(col_hbm, w_hbm, deg_hbm, col_v, w_v, zb_v, deg_row, deg_sh):
    cid = lax.axis_index("c")
    sid = lax.axis_index("s")
    wid = sid * NC + cid

    def zfill(i, _):
        zb_v[pl.ds(i * 16, 16)] = jnp.zeros((16,), jnp.float32)
        return 0
    lax.fori_loop(0, ROWS_PER_SUB // 16, zfill, 0)
    pltpu.sync_copy(zb_v, deg_sh.at[pl.ds(sid * ROWS_PER_SUB, ROWS_PER_SUB)])
    plsc.subcore_barrier()

    pltpu.sync_copy(col_hbm.at[sid], col_v)
    pltpu.sync_copy(w_hbm.at[sid], w_v)

    def dchunk(j, _):
        pltpu.sync_copy(w_v.at[j], deg_sh.at[col_v.at[j]], add=True)
        return 0
    lax.fori_loop(0, CH1, dchunk, 0)
    plsc.subcore_barrier()

    r0 = wid * ROWS_PER_TILE
    pltpu.sync_copy(deg_sh.at[pl.ds(r0, ROWS_PER_TILE)], deg_row)
    pltpu.sync_copy(deg_row, deg_hbm.at[pl.ds(r0, ROWS_PER_TILE)])


@functools.cache
def _k1_build():
    return pl.kernel(
        _k1_body,
        out_type=jax.ShapeDtypeStruct((NP,), jnp.float32),
        mesh=_mesh(),
        scratch_types=[
            pltpu.VMEM((CH1, 128), jnp.int32),
            pltpu.VMEM((CH1, 128), jnp.float32),
            pltpu.VMEM((ROWS_PER_SUB,), jnp.float32),
            pltpu.VMEM((ROWS_PER_TILE,), jnp.float32),
            pltpu.VMEM_SHARED((NP,), jnp.float32),
        ],
    )


def _k1(col16, w16):
    return _k1_build()(col16, w16)


def _k2_body(deg_ref, xf_ref, u_ref, db_ref):
    dinv = lax.rsqrt(deg_ref[...] + 1.0)
    db = jnp.broadcast_to(dinv, (NP, W))
    db_ref[...] = db
    u_ref[...] = db * xf_ref[...]


def _k2(deg, xf):
    return pl.pallas_call(
        _k2_body,
        out_shape=(jax.ShapeDtypeStruct((NP, W), jnp.float32),
                   jax.ShapeDtypeStruct((NP, W), jnp.float32)),
    )(deg.reshape(NP, 1), xf)


def _k3_body(row_hbm, col_hbm, w_hbm, u_hbm, sp_hbm,
             row_v, col_v, w_v, rows_v, zb_v, s_sh, sem):
    cid = lax.axis_index("c")
    sid = lax.axis_index("s")
    wid = sid * NC + cid

    def zfill(i, _):
        zb_v[pl.ds(i * 16, 16), :] = jnp.zeros((16, W), jnp.float32)
        return 0
    lax.fori_loop(0, 4, zfill, 0)

    def zcopy(i, _):
        pltpu.sync_copy(zb_v, s_sh.at[pl.ds(sid * ROWS_PER_SUB + i * 64, 64)])
        return 0
    lax.fori_loop(0, ROWS_PER_SUB // 64, zcopy, 0)
    plsc.subcore_barrier()

    pltpu.sync_copy(row_hbm.at[wid], row_v)
    pltpu.sync_copy(col_hbm.at[wid], col_v)
    pltpu.sync_copy(w_hbm.at[wid], w_v)

    def chunk(j, _):
        pltpu.async_copy(u_hbm.at[row_v.at[j]], rows_v, sem).wait()

        def escale(g, _):
            wv = w_v[j, pl.ds(g * 16, 16)]
            for k in range(16):
                e = g * 16 + k
                s = lax.broadcast_in_dim(wv[k], (16,), ())
                rows_v[e, pl.ds(0, 16)] = rows_v[e, pl.ds(0, 16)] * s
                rows_v[e, pl.ds(16, 16)] = rows_v[e, pl.ds(16, 16)] * s
            return 0
        lax.fori_loop(0, 8, escale, 0)

        pltpu.sync_copy(rows_v, s_sh.at[col_v.at[j]], add=True)
        return 0
    lax.fori_loop(0, CH3, chunk, 0)
    plsc.subcore_barrier()

    pltpu.sync_copy(s_sh.at[pl.ds(sid * ROWS_PER_SUB, ROWS_PER_SUB)],
                    sp_hbm.at[cid, pl.ds(sid * ROWS_PER_SUB, ROWS_PER_SUB)])


@functools.cache
def _k3_build():
    return pl.kernel(
        _k3_body,
        out_type=jax.ShapeDtypeStruct((NC, NP, W), jnp.float32),
        mesh=_mesh(),
        compiler_params=pltpu.CompilerParams(use_tc_tiling_on_sc=False),
        scratch_types=[
            pltpu.VMEM((CH3, 128), jnp.int32),
            pltpu.VMEM((CH3, 128), jnp.int32),
            pltpu.VMEM((CH3, 128), jnp.float32),
            pltpu.VMEM((128, W), jnp.float32),
            pltpu.VMEM((64, W), jnp.float32),
            pltpu.VMEM_SHARED((NP, W), jnp.float32),
            pltpu.SemaphoreType.DMA,
        ],
    )


def _k3(row32, col32, w32, u):
    return _k3_build()(row32, col32, w32, u)


def _gru_body(sp_ref, u_ref, db_ref, att_ref,
              czw_ref, czb_ref, lzw_ref, lzb_ref,
              crw_ref, crb_ref, lrw_ref, lrb_ref,
              chw_ref, chb_ref, lhw_ref, lhb_ref, out_ref):
    f32 = jnp.float32

    def dg(x, m):
        return lax.dot_general(x, m, (((1,), (1,)), ((), ())),
                               preferred_element_type=f32)

    Y = db_ref[...] * (sp_ref[0] + sp_ref[1] + u_ref[...])

    att = att_ref[...]
    ex = jnp.exp(att - jnp.max(att, axis=1, keepdims=True))
    probs = ex / jnp.sum(ex, axis=1, keepdims=True)

    def fold(lw_ref, lb_ref, cw_ref, cb_ref):
        A = lw_ref[:, 0:OUT]
        B = lw_ref[:, OUT:2 * OUT]
        M = lax.dot_general(A, cw_ref[...], (((1,), (0,)), ((), ())),
                            preferred_element_type=f32)
        b = dg(cb_ref[...], A) + lb_ref[...]
        return M, B, b

    Mz, Bz, bz = fold(lzw_ref, lzb_ref, czw_ref, czb_ref)
    Mr, Br, br = fold(lrw_ref, lrb_ref, crw_ref, crb_ref)
    Mh, Bh, bh = fold(lhw_ref, lhb_ref, chw_ref, chb_ref)

    H = jnp.zeros((BN, OUT), f32)
    acc = jnp.zeros((BN, OUT), f32)
    for p in range(PERIODS):
        Yp = Y[:, 2 * p:2 * p + 2]
        Z = jax.nn.sigmoid(dg(Yp, Mz) + dg(H, Bz) + bz)
        R = jax.nn.sigmoid(dg(Yp, Mr) + dg(H, Br) + br)
        Ht = jnp.tanh(dg(Yp, Mh) + dg(H * R, Bh) + bh)
        H = Z * H + (1.0 - Z) * Ht
        acc = acc + probs[0, p] * H
    out_ref[...] = acc


def kernel(X, edge_index, edge_weight, attention,
           conv_z_W, conv_z_b, lin_z_W, lin_z_b,
           conv_r_W, conv_r_b, lin_r_W, lin_r_b,
           conv_h_W, conv_h_b, lin_h_W, lin_h_b):
    f32 = jnp.float32
    ei = edge_index.astype(jnp.int32)
    w = edge_weight.astype(f32)
    pad = EP - E
    rowp = jnp.concatenate([ei[0], jnp.zeros((pad,), jnp.int32)])
    colp = jnp.concatenate([ei[1], jnp.zeros((pad,), jnp.int32)])
    wp = jnp.concatenate([w, jnp.zeros((pad,), f32)])

    col16 = colp.reshape(NS, CH1, 128)
    w16 = wp.reshape(NS, CH1, 128)
    row32 = rowp.reshape(NC * NS, CH3, 128)
    col32 = colp.reshape(NC * NS, CH3, 128)
    w32 = wp.reshape(NC * NS, CH3, 128)

    Xf = jnp.transpose(X[0], (0, 2, 1)).reshape(N, 2 * PERIODS)
    Xf = jnp.pad(Xf, ((0, NP - N), (0, W - 2 * PERIODS)))

    deg = _k1(col16, w16)
    U, dB = _k2(deg, Xf)
    SP = _k3(row32, col32, w32, U)

    full = lambda s: pl.BlockSpec(s, lambda i: (0,) * len(s))
    out = pl.pallas_call(
        _gru_body,
        grid=(NP // BN,),
        in_specs=[
            pl.BlockSpec((NC, BN, W), lambda i: (0, i, 0)),
            pl.BlockSpec((BN, W), lambda i: (i, 0)),
            pl.BlockSpec((BN, W), lambda i: (i, 0)),
            full((1, PERIODS)),
            full((OUT, 2)), full((1, OUT)), full((OUT, 2 * OUT)), full((1, OUT)),
            full((OUT, 2)), full((1, OUT)), full((OUT, 2 * OUT)), full((1, OUT)),
            full((OUT, 2)), full((1, OUT)), full((OUT, 2 * OUT)), full((1, OUT)),
        ],
        out_specs=pl.BlockSpec((BN, OUT), lambda i: (i, 0)),
        out_shape=jax.ShapeDtypeStruct((NP, OUT), f32),
    )(SP, U, dB, attention.reshape(1, PERIODS),
      conv_z_W, conv_z_b.reshape(1, OUT), lin_z_W, lin_z_b.reshape(1, OUT),
      conv_r_W, conv_r_b.reshape(1, OUT), lin_r_W, lin_r_b.reshape(1, OUT),
      conv_h_W, conv_h_b.reshape(1, OUT), lin_h_W, lin_h_b.reshape(1, OUT))

    return out[jnp.newaxis, :N, :]


# speedup vs baseline: 168.3085x; 1.1209x over previous
import functools

import jax
import jax.numpy as jnp
from jax import lax
from jax.experimental import pallas as pl
from jax.experimental.pallas import tpu as pltpu
from jax.experimental.pallas import tpu_sc as plsc

N = 10000
E = 160000
PERIODS = 12
OUT = 64
W = 32
NP = 10240
EP = 163840
NC = 2
NS = 16
ROWS_PER_TILE = NP // (NC * NS)
ROWS_PER_SUB = NP // NS
CH1 = EP // NS // 128
CH3 = EP // (NC * NS) // 128
BN = 2048
NB3 = 4

@functools.cache
def _mesh():
    return plsc.VectorSubcoreMesh(core_axis_name="c", subcore_axis_name="s",
                                  num_cores=NC, num_subcores=NS)


def _k1_body(col_hbm, w_hbm, deg_hbm, col_v, w_v, zb_v, deg_row, deg_sh):
    cid = lax.axis_index("c")
    sid = lax.axis_index("s")
    wid = sid * NC + cid

    def zfill(i, _):
        zb_v[pl.ds(i * 16, 16)] = jnp.zeros((16,), jnp.float32)
        return 0
    lax.fori_loop(0, ROWS_PER_SUB // 16, zfill, 0)
    pltpu.sync_copy(zb_v, deg_sh.at[pl.ds(sid * ROWS_PER_SUB, ROWS_PER_SUB)])
    plsc.subcore_barrier()

    pltpu.sync_copy(col_hbm.at[sid], col_v)
    pltpu.sync_copy(w_hbm.at[sid], w_v)

    def dchunk(j, _):
        pltpu.sync_copy(w_v.at[j], deg_sh.at[col_v.at[j]], add=True)
        return 0
    lax.fori_loop(0, CH1, dchunk, 0)
    plsc.subcore_barrier()

    r0 = wid * ROWS_PER_TILE
    pltpu.sync_copy(deg_sh.at[pl.ds(r0, ROWS_PER_TILE)], deg_row)
    pltpu.sync_copy(deg_row, deg_hbm.at[pl.ds(r0, ROWS_PER_TILE)])


@functools.cache
def _k1_build():
    return pl.kernel(
        _k1_body,
        out_type=jax.ShapeDtypeStruct((NP,), jnp.float32),
        mesh=_mesh(),
        scratch_types=[
            pltpu.VMEM((CH1, 128), jnp.int32),
            pltpu.VMEM((CH1, 128), jnp.float32),
            pltpu.VMEM((ROWS_PER_SUB,), jnp.float32),
            pltpu.VMEM((ROWS_PER_TILE,), jnp.float32),
            pltpu.VMEM_SHARED((NP,), jnp.float32),
        ],
    )


def _k1(col16, w16):
    return _k1_build()(col16, w16)


def _k2_body(deg_ref, xf_ref, u_ref, db_ref):
    dinv = lax.rsqrt(deg_ref[...] + 1.0)
    db = jnp.broadcast_to(dinv, (NP, W))
    db_ref[...] = db
    u_ref[...] = db * xf_ref[...]


def _k2(deg, xf):
    return pl.pallas_call(
        _k2_body,
        out_shape=(jax.ShapeDtypeStruct((NP, W), jnp.float32),
                   jax.ShapeDtypeStruct((NP, W), jnp.float32)),
    )(deg.reshape(NP, 1), xf)


def _k3_body(row_hbm, col_hbm, w_hbm, u_hbm, sp_hbm,
             row_v, col_v, w_v, rows_v, zb_v, s_sh, *sems):
    cid = lax.axis_index("c")
    sid = lax.axis_index("s")
    wid = sid * NC + cid

    def zfill(i, _):
        zb_v[pl.ds(i * 16, 16), :] = jnp.zeros((16, W), jnp.float32)
        return 0
    lax.fori_loop(0, 4, zfill, 0)

    def zcopy(i, _):
        pltpu.sync_copy(zb_v, s_sh.at[pl.ds(sid * ROWS_PER_SUB + i * 64, 64)])
        return 0
    lax.fori_loop(0, ROWS_PER_SUB // 64, zcopy, 0)
    plsc.subcore_barrier()

    pltpu.sync_copy(row_hbm.at[wid], row_v)
    pltpu.sync_copy(col_hbm.at[wid], col_v)
    pltpu.sync_copy(w_hbm.at[wid], w_v)

    for b in range(NB3):
        pltpu.async_copy(u_hbm.at[row_v.at[b]], rows_v.at[b], sems[b])

    def outer(jo, _):
        for b in range(NB3):
            j = jo * NB3 + b
            pltpu.make_async_copy(u_hbm.at[row_v.at[j]], rows_v.at[b],
                                  sems[b]).wait()

            def escale(g, _):
                wv = w_v[j, pl.ds(g * 16, 16)]
                for k in range(16):
                    e = g * 16 + k
                    s = lax.broadcast_in_dim(wv[k], (16,), ())
                    rows_v[b, e, pl.ds(0, 16)] = rows_v[b, e, pl.ds(0, 16)] * s
                    rows_v[b, e, pl.ds(16, 16)] = rows_v[b, e, pl.ds(16, 16)] * s
                return 0
            lax.fori_loop(0, 8, escale, 0)

            pltpu.sync_copy(rows_v.at[b], s_sh.at[col_v.at[j]], add=True)
            jn = j + NB3

            @pl.when(jn < CH3)
            def _fire():
                pltpu.async_copy(u_hbm.at[row_v.at[jn]], rows_v.at[b], sems[b])
        return 0
    lax.fori_loop(0, CH3 // NB3, outer, 0)
    plsc.subcore_barrier()

    pltpu.sync_copy(s_sh.at[pl.ds(sid * ROWS_PER_SUB, ROWS_PER_SUB)],
                    sp_hbm.at[cid, pl.ds(sid * ROWS_PER_SUB, ROWS_PER_SUB)])


@functools.cache
def _k3_build():
    return pl.kernel(
        _k3_body,
        out_type=jax.ShapeDtypeStruct((NC, NP, W), jnp.float32),
        mesh=_mesh(),
        compiler_params=pltpu.CompilerParams(use_tc_tiling_on_sc=False),
        scratch_types=[
            pltpu.VMEM((CH3, 128), jnp.int32),
            pltpu.VMEM((CH3, 128), jnp.int32),
            pltpu.VMEM((CH3, 128), jnp.float32),
            pltpu.VMEM((NB3, 128, W), jnp.float32),
            pltpu.VMEM((64, W), jnp.float32),
            pltpu.VMEM_SHARED((NP, W), jnp.float32),
        ] + [pltpu.SemaphoreType.DMA] * NB3,
    )


def _k3(row32, col32, w32, u):
    return _k3_build()(row32, col32, w32, u)


def _gru_body(sp_ref, u_ref, db_ref, att_ref,
              czw_ref, czb_ref, lzw_ref, lzb_ref,
              crw_ref, crb_ref, lrw_ref, lrb_ref,
              chw_ref, chb_ref, lhw_ref, lhb_ref, out_ref):
    f32 = jnp.float32

    def dg(x, m):
        return lax.dot_general(x, m, (((1,), (1,)), ((), ())),
                               preferred_element_type=f32)

    Y = db_ref[...] * (sp_ref[0] + sp_ref[1] + u_ref[...])

    att = att_ref[...]
    ex = jnp.exp(att - jnp.max(att, axis=1, keepdims=True))
    probs = ex / jnp.sum(ex, axis=1, keepdims=True)

    def fold(lw_ref, lb_ref, cw_ref, cb_ref):
        A = lw_ref[:, 0:OUT]
        B = lw_ref[:, OUT:2 * OUT]
        M = lax.dot_general(A, cw_ref[...], (((1,), (0,)), ((), ())),
                            preferred_element_type=f32)
        b = dg(cb_ref[...], A) + lb_ref[...]
        return M, B, b

    Mz, Bz, bz = fold(lzw_ref, lzb_ref, czw_ref, czb_ref)
    Mr, Br, br = fold(lrw_ref, lrb_ref, crw_ref, crb_ref)
    Mh, Bh, bh = fold(lhw_ref, lhb_ref, chw_ref, chb_ref)

    Y24 = Y[:, 0:2 * PERIODS]

    def sel(M, p):
        z = jnp.zeros((OUT, 1), f32)
        cols = [M[:, 0:1] if c == p else (M[:, 1:2] if c == PERIODS + p else z)
                for c in range(2 * PERIODS)]
        return jnp.concatenate(cols, axis=1)

    H = jnp.zeros((BN, OUT), f32)
    acc = jnp.zeros((BN, OUT), f32)
    for p in range(PERIODS):
        Z = jax.nn.sigmoid(dg(Y24, sel(Mz, p)) + dg(H, Bz) + bz)
        R = jax.nn.sigmoid(dg(Y24, sel(Mr, p)) + dg(H, Br) + br)
        Ht = jnp.tanh(dg(Y24, sel(Mh, p)) + dg(H * R, Bh) + bh)
        H = Z * H + (1.0 - Z) * Ht
        acc = acc + probs[0, p] * H
    out_ref[...] = acc


def kernel(X, edge_index, edge_weight, attention,
           conv_z_W, conv_z_b, lin_z_W, lin_z_b,
           conv_r_W, conv_r_b, lin_r_W, lin_r_b,
           conv_h_W, conv_h_b, lin_h_W, lin_h_b):
    f32 = jnp.float32
    ei = edge_index.astype(jnp.int32)
    w = edge_weight.astype(f32)
    pad = EP - E
    rowp = jnp.concatenate([ei[0], jnp.zeros((pad,), jnp.int32)])
    colp = jnp.concatenate([ei[1], jnp.zeros((pad,), jnp.int32)])
    wp = jnp.concatenate([w, jnp.zeros((pad,), f32)])

    col16 = colp.reshape(NS, CH1, 128)
    w16 = wp.reshape(NS, CH1, 128)
    row32 = rowp.reshape(NC * NS, CH3, 128)
    col32 = colp.reshape(NC * NS, CH3, 128)
    w32 = wp.reshape(NC * NS, CH3, 128)

    Xf = X[0].reshape(N, 2 * PERIODS)
    Xf = jnp.pad(Xf, ((0, NP - N), (0, W - 2 * PERIODS)))

    deg = _k1(col16, w16)
    U, dB = _k2(deg, Xf)
    SP = _k3(row32, col32, w32, U)

    full = lambda s: pl.BlockSpec(s, lambda i: (0,) * len(s))
    out = pl.pallas_call(
        _gru_body,
        grid=(NP // BN,),
        in_specs=[
            pl.BlockSpec((NC, BN, W), lambda i: (0, i, 0)),
            pl.BlockSpec((BN, W), lambda i: (i, 0)),
            pl.BlockSpec((BN, W), lambda i: (i, 0)),
            full((1, PERIODS)),
            full((OUT, 2)), full((1, OUT)), full((OUT, 2 * OUT)), full((1, OUT)),
            full((OUT, 2)), full((1, OUT)), full((OUT, 2 * OUT)), full((1, OUT)),
            full((OUT, 2)), full((1, OUT)), full((OUT, 2 * OUT)), full((1, OUT)),
        ],
        out_specs=pl.BlockSpec((BN, OUT), lambda i: (i, 0)),
        out_shape=jax.ShapeDtypeStruct((NP, OUT), f32),
    )(SP, U, dB, attention.reshape(1, PERIODS),
      conv_z_W, conv_z_b.reshape(1, OUT), lin_z_W, lin_z_b.reshape(1, OUT),
      conv_r_W, conv_r_b.reshape(1, OUT), lin_r_W, lin_r_b.reshape(1, OUT),
      conv_h_W, conv_h_b.reshape(1, OUT), lin_h_W, lin_h_b.reshape(1, OUT))

    return out[jnp.newaxis, :N, :]


# speedup vs baseline: 207.1556x; 1.2308x over previous
import functools

import jax
import jax.numpy as jnp
from jax import lax
from jax.experimental import pallas as pl
from jax.experimental.pallas import tpu as pltpu
from jax.experimental.pallas import tpu_sc as plsc

N = 10000
E = 160000
PERIODS = 12
OUT = 64
W = 32
NP = 10240
EP = 163840
NC = 2
NS = 16
ROWS_PER_TILE = NP // (NC * NS)
ROWS_PER_SUB = NP // NS
CH1 = EP // NS // 128
CH3 = EP // (NC * NS) // 128
BN = 2048
NB3 = 4

@functools.cache
def _mesh():
    return plsc.VectorSubcoreMesh(core_axis_name="c", subcore_axis_name="s",
                                  num_cores=NC, num_subcores=NS)


def _k1_body(col_hbm, w_hbm, deg_hbm, col_v, w_v, zb_v, deg_row, deg_sh):
    cid = lax.axis_index("c")
    sid = lax.axis_index("s")
    wid = sid * NC + cid

    def zfill(i, _):
        zb_v[pl.ds(i * 16, 16)] = jnp.zeros((16,), jnp.float32)
        return 0
    lax.fori_loop(0, ROWS_PER_SUB // 16, zfill, 0)
    pltpu.sync_copy(zb_v, deg_sh.at[pl.ds(sid * ROWS_PER_SUB, ROWS_PER_SUB)])
    plsc.subcore_barrier()

    pltpu.sync_copy(col_hbm.at[sid], col_v)
    pltpu.sync_copy(w_hbm.at[sid], w_v)

    def dchunk(j, _):
        pltpu.sync_copy(w_v.at[j], deg_sh.at[col_v.at[j]], add=True)
        return 0
    lax.fori_loop(0, CH1, dchunk, 0)
    plsc.subcore_barrier()

    r0 = wid * ROWS_PER_TILE
    pltpu.sync_copy(deg_sh.at[pl.ds(r0, ROWS_PER_TILE)], deg_row)
    pltpu.sync_copy(deg_row, deg_hbm.at[pl.ds(r0, ROWS_PER_TILE)])


@functools.cache
def _k1_build():
    return pl.kernel(
        _k1_body,
        out_type=jax.ShapeDtypeStruct((NP,), jnp.float32),
        mesh=_mesh(),
        scratch_types=[
            pltpu.VMEM((CH1, 128), jnp.int32),
            pltpu.VMEM((CH1, 128), jnp.float32),
            pltpu.VMEM((ROWS_PER_SUB,), jnp.float32),
            pltpu.VMEM((ROWS_PER_TILE,), jnp.float32),
            pltpu.VMEM_SHARED((NP,), jnp.float32),
        ],
    )


def _k1(col16, w16):
    return _k1_build()(col16, w16)


def _k2_body(deg_ref, xf_ref, u_ref, db_ref):
    dinv = lax.rsqrt(deg_ref[...] + 1.0)
    db = jnp.broadcast_to(dinv, (NP, W))
    db_ref[...] = db
    u_ref[...] = db * xf_ref[...]


def _k2(deg, xf):
    return pl.pallas_call(
        _k2_body,
        out_shape=(jax.ShapeDtypeStruct((NP, W), jnp.float32),
                   jax.ShapeDtypeStruct((NP, W), jnp.float32)),
    )(deg.reshape(NP, 1), xf)


def _k3_body(row_hbm, col_hbm, w_hbm, u_hbm, sp_hbm,
             row_v, col_v, w_v, rows_v, zb_v, s_sh, *sems):
    cid = lax.axis_index("c")
    sid = lax.axis_index("s")
    wid = sid * NC + cid

    def zfill(i, _):
        zb_v[pl.ds(i * 16, 16), :] = jnp.zeros((16, W), jnp.float32)
        return 0
    lax.fori_loop(0, 4, zfill, 0)

    def zcopy(i, _):
        pltpu.sync_copy(zb_v, s_sh.at[pl.ds(sid * ROWS_PER_SUB + i * 64, 64)])
        return 0
    lax.fori_loop(0, ROWS_PER_SUB // 64, zcopy, 0)
    plsc.subcore_barrier()

    pltpu.sync_copy(row_hbm.at[wid], row_v)
    pltpu.sync_copy(col_hbm.at[wid], col_v)
    pltpu.sync_copy(w_hbm.at[wid], w_v)

    for b in range(NB3):
        pltpu.async_copy(u_hbm.at[row_v.at[b]], rows_v.at[b], sems[b])

    def outer(jo, _):
        for b in range(NB3):
            j = jo * NB3 + b
            pltpu.make_async_copy(u_hbm.at[row_v.at[j]], rows_v.at[b],
                                  sems[b]).wait()

            def escale(g, _):
                wv = w_v[j, pl.ds(g * 16, 16)]
                for k in range(16):
                    e = g * 16 + k
                    s = lax.broadcast_in_dim(wv[k], (16,), ())
                    rows_v[b, e, pl.ds(0, 16)] = rows_v[b, e, pl.ds(0, 16)] * s
                    rows_v[b, e, pl.ds(16, 16)] = rows_v[b, e, pl.ds(16, 16)] * s
                return 0
            lax.fori_loop(0, 8, escale, 0)

            pltpu.sync_copy(rows_v.at[b], s_sh.at[col_v.at[j]], add=True)
            jn = j + NB3

            @pl.when(jn < CH3)
            def _fire():
                pltpu.async_copy(u_hbm.at[row_v.at[jn]], rows_v.at[b], sems[b])
        return 0
    lax.fori_loop(0, CH3 // NB3, outer, 0)
    plsc.subcore_barrier()

    pltpu.sync_copy(s_sh.at[pl.ds(sid * ROWS_PER_SUB, ROWS_PER_SUB)],
                    sp_hbm.at[cid, pl.ds(sid * ROWS_PER_SUB, ROWS_PER_SUB)])


@functools.cache
def _k3_build():
    return pl.kernel(
        _k3_body,
        out_type=jax.ShapeDtypeStruct((NC, NP, W), jnp.float32),
        mesh=_mesh(),
        compiler_params=pltpu.CompilerParams(use_tc_tiling_on_sc=False),
        scratch_types=[
            pltpu.VMEM((CH3, 128), jnp.int32),
            pltpu.VMEM((CH3, 128), jnp.int32),
            pltpu.VMEM((CH3, 128), jnp.float32),
            pltpu.VMEM((NB3, 128, W), jnp.float32),
            pltpu.VMEM((64, W), jnp.float32),
            pltpu.VMEM_SHARED((NP, W), jnp.float32),
        ] + [pltpu.SemaphoreType.DMA] * NB3,
    )


def _k3(row32, col32, w32, u):
    return _k3_build()(row32, col32, w32, u)


def _gru_body(sp_ref, u_ref, db_ref, att_ref,
              czw_ref, czb_ref, lzw_ref, lzb_ref,
              crw_ref, crb_ref, lrw_ref, lrb_ref,
              chw_ref, chb_ref, lhw_ref, lhb_ref, out_ref):
    f32 = jnp.float32

    def dg(x, m):
        return lax.dot_general(x, m, (((1,), (1,)), ((), ())),
                               preferred_element_type=f32)

    Y = db_ref[...] * (sp_ref[0] + sp_ref[1] + u_ref[...])

    att = att_ref[...]
    ex = jnp.exp(att - jnp.max(att, axis=1, keepdims=True))
    probs = ex / jnp.sum(ex, axis=1, keepdims=True)

    def fold(lw_ref, lb_ref, cw_ref, cb_ref):
        A = lw_ref[:, 0:OUT]
        B = lw_ref[:, OUT:2 * OUT]
        M = lax.dot_general(A, cw_ref[...], (((1,), (0,)), ((), ())),
                            preferred_element_type=f32)
        b = dg(cb_ref[...], A) + lb_ref[...]
        return M, B, b

    Mz, Bz, bz = fold(lzw_ref, lzb_ref, czw_ref, czb_ref)
    Mr, Br, br = fold(lrw_ref, lrb_ref, crw_ref, crb_ref)
    Mh, Bh, bh = fold(lhw_ref, lhb_ref, chw_ref, chb_ref)

    Y24 = Y[:, 0:2 * PERIODS]

    def sel(M, p):
        z = jnp.zeros((OUT, 1), f32)
        cols = [M[:, 0:1] if c == p else (M[:, 1:2] if c == PERIODS + p else z)
                for c in range(2 * PERIODS)]
        return jnp.concatenate(cols, axis=1)

    H = jnp.zeros((BN, OUT), f32)
    acc = jnp.zeros((BN, OUT), f32)
    for p in range(PERIODS):
        sg = lambda x: 1.0 / (1.0 + jnp.exp(-x))
        Z = sg(dg(Y24, sel(Mz, p)) + dg(H, Bz) + bz)
        R = sg(dg(Y24, sel(Mr, p)) + dg(H, Br) + br)
        Ht = jnp.tanh(dg(Y24, sel(Mh, p)) + dg(H * R, Bh) + bh)
        H = Z * H + (1.0 - Z) * Ht
        acc = acc + probs[0, p] * H
    out_ref[...] = acc


def kernel(X, edge_index, edge_weight, attention,
           conv_z_W, conv_z_b, lin_z_W, lin_z_b,
           conv_r_W, conv_r_b, lin_r_W, lin_r_b,
           conv_h_W, conv_h_b, lin_h_W, lin_h_b):
    f32 = jnp.float32
    ei = edge_index.astype(jnp.int32)
    w = edge_weight.astype(f32)
    pad = EP - E
    spread = (jnp.arange(pad, dtype=jnp.int32) * 7) % NP
    rowp = jnp.concatenate([ei[0], spread])
    colp = jnp.concatenate([ei[1], spread])
    wp = jnp.concatenate([w, jnp.zeros((pad,), f32)])

    col16 = colp.reshape(NS, CH1, 128)
    w16 = wp.reshape(NS, CH1, 128)
    row32 = rowp.reshape(NC * NS, CH3, 128)
    col32 = colp.reshape(NC * NS, CH3, 128)
    w32 = wp.reshape(NC * NS, CH3, 128)

    Xf = X[0].reshape(N, 2 * PERIODS)
    Xf = jnp.pad(Xf, ((0, NP - N), (0, W - 2 * PERIODS)))

    deg = _k1(col16, w16)
    U, dB = _k2(deg, Xf)
    SP = _k3(row32, col32, w32, U)

    full = lambda s: pl.BlockSpec(s, lambda i: (0,) * len(s))
    out = pl.pallas_call(
        _gru_body,
        grid=(NP // BN,),
        in_specs=[
            pl.BlockSpec((NC, BN, W), lambda i: (0, i, 0)),
            pl.BlockSpec((BN, W), lambda i: (i, 0)),
            pl.BlockSpec((BN, W), lambda i: (i, 0)),
            full((1, PERIODS)),
            full((OUT, 2)), full((1, OUT)), full((OUT, 2 * OUT)), full((1, OUT)),
            full((OUT, 2)), full((1, OUT)), full((OUT, 2 * OUT)), full((1, OUT)),
            full((OUT, 2)), full((1, OUT)), full((OUT, 2 * OUT)), full((1, OUT)),
        ],
        out_specs=pl.BlockSpec((BN, OUT), lambda i: (i, 0)),
        out_shape=jax.ShapeDtypeStruct((NP, OUT), f32),
    )(SP, U, dB, attention.reshape(1, PERIODS),
      conv_z_W, conv_z_b.reshape(1, OUT), lin_z_W, lin_z_b.reshape(1, OUT),
      conv_r_W, conv_r_b.reshape(1, OUT), lin_r_W, lin_r_b.reshape(1, OUT),
      conv_h_W, conv_h_b.reshape(1, OUT), lin_h_W, lin_h_b.reshape(1, OUT))

    return out[jnp.newaxis, :N, :]


# speedup vs baseline: 213.8227x; 1.0322x over previous
import functools

import jax
import jax.numpy as jnp
from jax import lax
from jax.experimental import pallas as pl
from jax.experimental.pallas import tpu as pltpu
from jax.experimental.pallas import tpu_sc as plsc

N = 10000
E = 160000
PERIODS = 12
OUT = 64
W = 32
NP = 10240
EP = 163840
NC = 2
NS = 16
ROWS_PER_TILE = NP // (NC * NS)
ROWS_PER_SUB = NP // NS
CH1 = EP // NS // 128
CH3 = EP // (NC * NS) // 128
BN = 2048
NB3 = 4

@functools.cache
def _mesh():
    return plsc.VectorSubcoreMesh(core_axis_name="c", subcore_axis_name="s",
                                  num_cores=NC, num_subcores=NS)


def _k1_body(col_hbm, w_hbm, deg_hbm, col_v, w_v, zb_v, deg_row, deg_sh,
             dsem):
    cid = lax.axis_index("c")
    sid = lax.axis_index("s")
    wid = sid * NC + cid

    def zfill(i, _):
        zb_v[pl.ds(i * 16, 16)] = jnp.zeros((16,), jnp.float32)
        return 0
    lax.fori_loop(0, ROWS_PER_SUB // 16, zfill, 0)
    pltpu.sync_copy(zb_v, deg_sh.at[pl.ds(sid * ROWS_PER_SUB, ROWS_PER_SUB)])
    plsc.subcore_barrier()

    pltpu.sync_copy(col_hbm.at[sid], col_v)
    pltpu.sync_copy(w_hbm.at[sid], w_v)

    def dfire(j, _):
        pltpu.async_copy(w_v.at[j], deg_sh.at[col_v.at[j]], dsem, add=True)
        return 0
    lax.fori_loop(0, CH1, dfire, 0)

    def ddrain(j, _):
        pltpu.make_async_copy(w_v.at[j], deg_sh.at[col_v.at[j]], dsem).wait()
        return 0
    lax.fori_loop(0, CH1, ddrain, 0)
    plsc.subcore_barrier()

    r0 = wid * ROWS_PER_TILE
    pltpu.sync_copy(deg_sh.at[pl.ds(r0, ROWS_PER_TILE)], deg_row)
    pltpu.sync_copy(deg_row, deg_hbm.at[pl.ds(r0, ROWS_PER_TILE)])


@functools.cache
def _k1_build():
    return pl.kernel(
        _k1_body,
        out_type=jax.ShapeDtypeStruct((NP,), jnp.float32),
        mesh=_mesh(),
        scratch_types=[
            pltpu.VMEM((CH1, 128), jnp.int32),
            pltpu.VMEM((CH1, 128), jnp.float32),
            pltpu.VMEM((ROWS_PER_SUB,), jnp.float32),
            pltpu.VMEM((ROWS_PER_TILE,), jnp.float32),
            pltpu.VMEM_SHARED((NP,), jnp.float32),
            pltpu.SemaphoreType.DMA,
        ],
    )


def _k1(col16, w16):
    return _k1_build()(col16, w16)


EROWS = E // 128
EPROWS = EP // 128


def _prep_body(ei_ref, w_ref, x_ref, row_ref, col_ref, wp_ref, xf_ref):
    i32, f32 = jnp.int32, jnp.float32
    row_ref[pl.ds(0, EROWS), :] = ei_ref[0]
    col_ref[pl.ds(0, EROWS), :] = ei_ref[1]
    wp_ref[pl.ds(0, EROWS), :] = w_ref[...]
    npad = EPROWS - EROWS
    flat = (jax.lax.broadcasted_iota(i32, (npad, 128), 0) * 128
            + jax.lax.broadcasted_iota(i32, (npad, 128), 1))
    spread = (flat * 7) % NP
    row_ref[pl.ds(EROWS, npad), :] = spread
    col_ref[pl.ds(EROWS, npad), :] = spread
    wp_ref[pl.ds(EROWS, npad), :] = jnp.zeros((npad, 128), f32)
    xf_ref[...] = jnp.zeros((NP, W), f32)
    xf_ref[pl.ds(0, N), pl.ds(0, 2 * PERIODS)] = x_ref[...]


def _prep(ei, w, x24):
    i32, f32 = jnp.int32, jnp.float32
    return pl.pallas_call(
        _prep_body,
        out_shape=(jax.ShapeDtypeStruct((EPROWS, 128), i32),
                   jax.ShapeDtypeStruct((EPROWS, 128), i32),
                   jax.ShapeDtypeStruct((EPROWS, 128), f32),
                   jax.ShapeDtypeStruct((NP, W), f32)),
    )(ei.reshape(2, EROWS, 128), w.reshape(EROWS, 128), x24)


def _k2_body(deg_ref, xf_ref, u_ref, db_ref):
    dinv = lax.rsqrt(deg_ref[...] + 1.0)
    db = jnp.broadcast_to(dinv, (NP, W))
    db_ref[...] = db
    u_ref[...] = db * xf_ref[...]


def _k2(deg, xf):
    return pl.pallas_call(
        _k2_body,
        out_shape=(jax.ShapeDtypeStruct((NP, W), jnp.float32),
                   jax.ShapeDtypeStruct((NP, W), jnp.float32)),
    )(deg.reshape(NP, 1), xf)


def _k3_body(row_hbm, col_hbm, w_hbm, u_hbm, sp_hbm,
             row_v, col_v, w_v, rows_v, zb_v, s_sh, *sems):
    cid = lax.axis_index("c")
    sid = lax.axis_index("s")
    wid = sid * NC + cid

    def zfill(i, _):
        zb_v[pl.ds(i * 16, 16), :] = jnp.zeros((16, W), jnp.float32)
        return 0
    lax.fori_loop(0, 4, zfill, 0)

    def zcopy(i, _):
        pltpu.sync_copy(zb_v, s_sh.at[pl.ds(sid * ROWS_PER_SUB + i * 64, 64)])
        return 0
    lax.fori_loop(0, ROWS_PER_SUB // 64, zcopy, 0)
    plsc.subcore_barrier()

    pltpu.sync_copy(row_hbm.at[wid], row_v)
    pltpu.sync_copy(col_hbm.at[wid], col_v)
    pltpu.sync_copy(w_hbm.at[wid], w_v)

    for b in range(NB3):
        pltpu.async_copy(u_hbm.at[row_v.at[b]], rows_v.at[b], sems[b])

    def outer(jo, _):
        for b in range(NB3):
            j = jo * NB3 + b
            pltpu.make_async_copy(u_hbm.at[row_v.at[j]], rows_v.at[b],
                                  sems[b]).wait()

            def escale(g, _):
                wv = w_v[j, pl.ds(g * 16, 16)]
                for k in range(16):
                    e = g * 16 + k
                    s = lax.broadcast_in_dim(wv[k], (16,), ())
                    rows_v[b, e, pl.ds(0, 16)] = rows_v[b, e, pl.ds(0, 16)] * s
                    rows_v[b, e, pl.ds(16, 16)] = rows_v[b, e, pl.ds(16, 16)] * s
                return 0
            lax.fori_loop(0, 8, escale, 0)

            pltpu.sync_copy(rows_v.at[b], s_sh.at[col_v.at[j]], add=True)
            jn = j + NB3

            @pl.when(jn < CH3)
            def _fire():
                pltpu.async_copy(u_hbm.at[row_v.at[jn]], rows_v.at[b], sems[b])
        return 0
    lax.fori_loop(0, CH3 // NB3, outer, 0)
    plsc.subcore_barrier()

    pltpu.sync_copy(s_sh.at[pl.ds(sid * ROWS_PER_SUB, ROWS_PER_SUB)],
                    sp_hbm.at[cid, pl.ds(sid * ROWS_PER_SUB, ROWS_PER_SUB)])


@functools.cache
def _k3_build():
    return pl.kernel(
        _k3_body,
        out_type=jax.ShapeDtypeStruct((NC, NP, W), jnp.float32),
        mesh=_mesh(),
        compiler_params=pltpu.CompilerParams(use_tc_tiling_on_sc=False),
        scratch_types=[
            pltpu.VMEM((CH3, 128), jnp.int32),
            pltpu.VMEM((CH3, 128), jnp.int32),
            pltpu.VMEM((CH3, 128), jnp.float32),
            pltpu.VMEM((NB3, 128, W), jnp.float32),
            pltpu.VMEM((64, W), jnp.float32),
            pltpu.VMEM_SHARED((NP, W), jnp.float32),
        ] + [pltpu.SemaphoreType.DMA] * NB3,
    )


def _k3(row32, col32, w32, u):
    return _k3_build()(row32, col32, w32, u)


def _gru_body(sp_ref, u_ref, db_ref, att_ref,
              czw_ref, czb_ref, lzw_ref, lzb_ref,
              crw_ref, crb_ref, lrw_ref, lrb_ref,
              chw_ref, chb_ref, lhw_ref, lhb_ref, out_ref):
    f32 = jnp.float32

    bf16 = jnp.bfloat16

    def dg(x, m):
        return lax.dot_general(x, m.astype(bf16), (((1,), (1,)), ((), ())),
                               preferred_element_type=f32)

    Y = db_ref[...] * (sp_ref[0] + sp_ref[1] + u_ref[...])

    att = att_ref[...]
    ex = jnp.exp(att - jnp.max(att, axis=1, keepdims=True))
    probs = ex / jnp.sum(ex, axis=1, keepdims=True)

    def fold(lw_ref, lb_ref, cw_ref, cb_ref):
        A = lw_ref[:, 0:OUT]
        B = lw_ref[:, OUT:2 * OUT]
        M = lax.dot_general(A, cw_ref[...], (((1,), (0,)), ((), ())),
                            preferred_element_type=f32)
        b = dg(cb_ref[...], A) + lb_ref[...]
        return M, B, b

    Mz, Bz, bz = fold(lzw_ref, lzb_ref, czw_ref, czb_ref)
    Mr, Br, br = fold(lrw_ref, lrb_ref, crw_ref, crb_ref)
    Mh, Bh, bh = fold(lhw_ref, lhb_ref, chw_ref, chb_ref)

    Y24 = Y[:, 0:2 * PERIODS]

    def sel(M, p):
        z = jnp.zeros((OUT, 1), f32)
        cols = [M[:, 0:1] if c == p else (M[:, 1:2] if c == PERIODS + p else z)
                for c in range(2 * PERIODS)]
        return jnp.concatenate(cols, axis=1)

    H = jnp.zeros((BN, OUT), f32)
    acc = jnp.zeros((BN, OUT), f32)
    Y24b = Y24.astype(bf16)
    for p in range(PERIODS):
        sg = lambda x: 1.0 / (1.0 + jnp.exp(-x))
        Hb = H.astype(bf16)
        Z = sg(dg(Y24b, sel(Mz, p)) + dg(Hb, Bz) + bz)
        R = sg(dg(Y24b, sel(Mr, p)) + dg(Hb, Br) + br)
        Ht = jnp.tanh(dg(Y24b, sel(Mh, p)) + dg((H * R).astype(bf16), Bh) + bh)
        H = Z * H + (1.0 - Z) * Ht
        acc = acc + probs[0, p] * H
    out_ref[...] = acc


def kernel(X, edge_index, edge_weight, attention,
           conv_z_W, conv_z_b, lin_z_W, lin_z_b,
           conv_r_W, conv_r_b, lin_r_W, lin_r_b,
           conv_h_W, conv_h_b, lin_h_W, lin_h_b):
    f32 = jnp.float32
    ei = edge_index.astype(jnp.int32)
    w = edge_weight.astype(f32)
    rowp, colp, wp, Xf = _prep(ei, w, X[0].reshape(N, 2 * PERIODS))

    col16 = colp.reshape(NS, CH1, 128)
    w16 = wp.reshape(NS, CH1, 128)
    row32 = rowp.reshape(NC * NS, CH3, 128)
    col32 = colp.reshape(NC * NS, CH3, 128)
    w32 = wp.reshape(NC * NS, CH3, 128)

    deg = _k1(col16, w16)
    U, dB = _k2(deg, Xf)
    SP = _k3(row32, col32, w32, U)

    full = lambda s: pl.BlockSpec(s, lambda i: (0,) * len(s))
    out = pl.pallas_call(
        _gru_body,
        grid=(NP // BN,),
        in_specs=[
            pl.BlockSpec((NC, BN, W), lambda i: (0, i, 0)),
            pl.BlockSpec((BN, W), lambda i: (i, 0)),
            pl.BlockSpec((BN, W), lambda i: (i, 0)),
            full((1, PERIODS)),
            full((OUT, 2)), full((1, OUT)), full((OUT, 2 * OUT)), full((1, OUT)),
            full((OUT, 2)), full((1, OUT)), full((OUT, 2 * OUT)), full((1, OUT)),
            full((OUT, 2)), full((1, OUT)), full((OUT, 2 * OUT)), full((1, OUT)),
        ],
        out_specs=pl.BlockSpec((BN, OUT), lambda i: (i, 0)),
        out_shape=jax.ShapeDtypeStruct((N, OUT), f32),
    )(SP, U, dB, attention.reshape(1, PERIODS),
      conv_z_W, conv_z_b.reshape(1, OUT), lin_z_W, lin_z_b.reshape(1, OUT),
      conv_r_W, conv_r_b.reshape(1, OUT), lin_r_W, lin_r_b.reshape(1, OUT),
      conv_h_W, conv_h_b.reshape(1, OUT), lin_h_W, lin_h_b.reshape(1, OUT))

    return out[jnp.newaxis]


# speedup vs baseline: 227.1526x; 1.0623x over previous
import functools

import jax
import jax.numpy as jnp
from jax import lax
from jax.experimental import pallas as pl
from jax.experimental.pallas import tpu as pltpu
from jax.experimental.pallas import tpu_sc as plsc

N = 10000
E = 160000
PERIODS = 12
OUT = 64
W = 32
NP = 10240
EP = 163840
NC = 2
NS = 16
ROWS_PER_TILE = NP // (NC * NS)
ROWS_PER_SUB = NP // NS
CH1 = EP // NS // 128
CH3 = EP // (NC * NS) // 128
BN = 2048
NB3 = 4

@functools.cache
def _mesh():
    return plsc.VectorSubcoreMesh(core_axis_name="c", subcore_axis_name="s",
                                  num_cores=NC, num_subcores=NS)


def _k1_body(col_hbm, w_hbm, deg_hbm, col_v, w_v, zb_v, deg_row, deg_sh,
             dsem):
    cid = lax.axis_index("c")
    sid = lax.axis_index("s")
    wid = sid * NC + cid

    def zfill(i, _):
        zb_v[pl.ds(i * 16, 16)] = jnp.zeros((16,), jnp.float32)
        return 0
    lax.fori_loop(0, ROWS_PER_SUB // 16, zfill, 0)
    pltpu.sync_copy(zb_v, deg_sh.at[pl.ds(sid * ROWS_PER_SUB, ROWS_PER_SUB)])
    plsc.subcore_barrier()

    pltpu.sync_copy(col_hbm.at[sid], col_v)
    pltpu.sync_copy(w_hbm.at[sid], w_v)

    def dfire(j, _):
        pltpu.async_copy(w_v.at[j], deg_sh.at[col_v.at[j]], dsem, add=True)
        return 0
    lax.fori_loop(0, CH1, dfire, 0)

    def ddrain(j, _):
        pltpu.make_async_copy(w_v.at[j], deg_sh.at[col_v.at[j]], dsem).wait()
        return 0
    lax.fori_loop(0, CH1, ddrain, 0)
    plsc.subcore_barrier()

    r0 = wid * ROWS_PER_TILE
    pltpu.sync_copy(deg_sh.at[pl.ds(r0, ROWS_PER_TILE)], deg_row)
    pltpu.sync_copy(deg_row, deg_hbm.at[pl.ds(r0, ROWS_PER_TILE)])


@functools.cache
def _k1_build():
    return pl.kernel(
        _k1_body,
        out_type=jax.ShapeDtypeStruct((NP,), jnp.float32),
        mesh=_mesh(),
        scratch_types=[
            pltpu.VMEM((CH1, 128), jnp.int32),
            pltpu.VMEM((CH1, 128), jnp.float32),
            pltpu.VMEM((ROWS_PER_SUB,), jnp.float32),
            pltpu.VMEM((ROWS_PER_TILE,), jnp.float32),
            pltpu.VMEM_SHARED((NP,), jnp.float32),
            pltpu.SemaphoreType.DMA,
        ],
    )


def _k1(col16, w16):
    return _k1_build()(col16, w16)


EROWS = E // 128
EPROWS = EP // 128


def _prep_body(ei_ref, w_ref, x_ref, row_ref, col_ref, wp_ref, xf_ref):
    i32, f32 = jnp.int32, jnp.float32
    row_ref[pl.ds(0, EROWS), :] = ei_ref[0]
    col_ref[pl.ds(0, EROWS), :] = ei_ref[1]
    wp_ref[pl.ds(0, EROWS), :] = w_ref[...]
    npad = EPROWS - EROWS
    flat = (jax.lax.broadcasted_iota(i32, (npad, 128), 0) * 128
            + jax.lax.broadcasted_iota(i32, (npad, 128), 1))
    spread = (flat * 7) % NP
    row_ref[pl.ds(EROWS, npad), :] = spread
    col_ref[pl.ds(EROWS, npad), :] = spread
    wp_ref[pl.ds(EROWS, npad), :] = jnp.zeros((npad, 128), f32)
    xf_ref[...] = jnp.zeros((NP, W), f32)
    xf_ref[pl.ds(0, N), pl.ds(0, 2 * PERIODS)] = x_ref[...]


def _prep(ei, w, x24):
    i32, f32 = jnp.int32, jnp.float32
    return pl.pallas_call(
        _prep_body,
        out_shape=(jax.ShapeDtypeStruct((EPROWS, 128), i32),
                   jax.ShapeDtypeStruct((EPROWS, 128), i32),
                   jax.ShapeDtypeStruct((EPROWS, 128), f32),
                   jax.ShapeDtypeStruct((NP, W), f32)),
    )(ei.reshape(2, EROWS, 128), w.reshape(EROWS, 128), x24)


def _k2_body(deg_ref, xf_ref, u_ref, db_ref):
    dinv = lax.rsqrt(deg_ref[...] + 1.0)
    db = jnp.broadcast_to(dinv, (NP, W))
    db_ref[...] = db
    u_ref[...] = db * xf_ref[...]


def _k2(deg, xf):
    return pl.pallas_call(
        _k2_body,
        out_shape=(jax.ShapeDtypeStruct((NP, W), jnp.float32),
                   jax.ShapeDtypeStruct((NP, W), jnp.float32)),
    )(deg.reshape(NP, 1), xf)


def _k3_body(row_hbm, col_hbm, w_hbm, u_hbm, sp_hbm,
             row_v, col_v, w_v, rows_v, zb_v, s_sh, *sems):
    cid = lax.axis_index("c")
    sid = lax.axis_index("s")
    wid = sid * NC + cid

    def zfill(i, _):
        zb_v[pl.ds(i * 16, 16), :] = jnp.zeros((16, W), jnp.float32)
        return 0
    lax.fori_loop(0, 4, zfill, 0)

    def zcopy(i, _):
        pltpu.sync_copy(zb_v, s_sh.at[pl.ds(sid * ROWS_PER_SUB + i * 64, 64)])
        return 0
    lax.fori_loop(0, ROWS_PER_SUB // 64, zcopy, 0)
    plsc.subcore_barrier()

    pltpu.sync_copy(row_hbm.at[wid], row_v)
    pltpu.sync_copy(col_hbm.at[wid], col_v)
    pltpu.sync_copy(w_hbm.at[wid], w_v)

    for b in range(NB3):
        pltpu.async_copy(u_hbm.at[row_v.at[b]], rows_v.at[b], sems[b])

    def outer(jo, _):
        for b in range(NB3):
            j = jo * NB3 + b
            pltpu.make_async_copy(u_hbm.at[row_v.at[j]], rows_v.at[b],
                                  sems[b]).wait()

            def escale(g, _):
                wv = w_v[j, pl.ds(g * 16, 16)]
                for k in range(16):
                    e = g * 16 + k
                    s = lax.broadcast_in_dim(wv[k], (16,), ())
                    rows_v[b, e, pl.ds(0, 16)] = rows_v[b, e, pl.ds(0, 16)] * s
                    rows_v[b, e, pl.ds(16, 16)] = rows_v[b, e, pl.ds(16, 16)] * s
                return 0
            lax.fori_loop(0, 8, escale, 0)

            pltpu.sync_copy(rows_v.at[b], s_sh.at[col_v.at[j]], add=True)
            jn = j + NB3

            @pl.when(jn < CH3)
            def _fire():
                pltpu.async_copy(u_hbm.at[row_v.at[jn]], rows_v.at[b], sems[b])
        return 0
    lax.fori_loop(0, CH3 // NB3, outer, 0)
    plsc.subcore_barrier()

    pltpu.sync_copy(s_sh.at[pl.ds(sid * ROWS_PER_SUB, ROWS_PER_SUB)],
                    sp_hbm.at[cid, pl.ds(sid * ROWS_PER_SUB, ROWS_PER_SUB)])


@functools.cache
def _k3_build():
    return pl.kernel(
        _k3_body,
        out_type=jax.ShapeDtypeStruct((NC, NP, W), jnp.float32),
        mesh=_mesh(),
        compiler_params=pltpu.CompilerParams(use_tc_tiling_on_sc=False),
        scratch_types=[
            pltpu.VMEM((CH3, 128), jnp.int32),
            pltpu.VMEM((CH3, 128), jnp.int32),
            pltpu.VMEM((CH3, 128), jnp.float32),
            pltpu.VMEM((NB3, 128, W), jnp.float32),
            pltpu.VMEM((64, W), jnp.float32),
            pltpu.VMEM_SHARED((NP, W), jnp.float32),
        ] + [pltpu.SemaphoreType.DMA] * NB3,
    )


def _k3(row32, col32, w32, u):
    return _k3_build()(row32, col32, w32, u)


def _gru_body(sp_ref, u_ref, db_ref, att_ref,
              czw_ref, czb_ref, lzw_ref, lzb_ref,
              crw_ref, crb_ref, lrw_ref, lrb_ref,
              chw_ref, chb_ref, lhw_ref, lhb_ref, out_ref):
    f32 = jnp.float32

    def dg(x, m):
        return lax.dot_general(x, m, (((1,), (1,)), ((), ())),
                               preferred_element_type=f32)

    Y = db_ref[...] * (sp_ref[0] + sp_ref[1] + u_ref[...])

    att = att_ref[...]
    ex = jnp.exp(att - jnp.max(att, axis=1, keepdims=True))
    probs = ex / jnp.sum(ex, axis=1, keepdims=True)

    def fold(lw_ref, lb_ref, cw_ref, cb_ref):
        A = lw_ref[:, 0:OUT]
        B = lw_ref[:, OUT:2 * OUT]
        M = lax.dot_general(A, cw_ref[...], (((1,), (0,)), ((), ())),
                            preferred_element_type=f32)
        b = dg(cb_ref[...], A) + lb_ref[...]
        return M, B, b

    Mz, Bz, bz = fold(lzw_ref, lzb_ref, czw_ref, czb_ref)
    Mr, Br, br = fold(lrw_ref, lrb_ref, crw_ref, crb_ref)
    Mh, Bh, bh = fold(lhw_ref, lhb_ref, chw_ref, chb_ref)

    Y24 = Y[:, 0:2 * PERIODS]

    def sel(M, p):
        a = jnp.pad(M[:, 0:1], ((0, 0), (p, 2 * PERIODS - 1 - p)))
        b = jnp.pad(M[:, 1:2], ((0, 0), (PERIODS + p, PERIODS - 1 - p)))
        return a + b

    H = jnp.zeros((BN, OUT), f32)
    acc = jnp.zeros((BN, OUT), f32)
    sg = lambda x: 0.5 * (jnp.tanh(0.5 * x) + 1.0)
    for p in range(PERIODS):
        Z = sg(dg(Y24, sel(Mz, p)) + dg(H, Bz) + bz)
        R = sg(dg(Y24, sel(Mr, p)) + dg(H, Br) + br)
        Ht = jnp.tanh(dg(Y24, sel(Mh, p)) + dg(H * R, Bh) + bh)
        H = Z * H + (1.0 - Z) * Ht
        acc = acc + probs[0, p] * H
    out_ref[...] = acc.T


def kernel(X, edge_index, edge_weight, attention,
           conv_z_W, conv_z_b, lin_z_W, lin_z_b,
           conv_r_W, conv_r_b, lin_r_W, lin_r_b,
           conv_h_W, conv_h_b, lin_h_W, lin_h_b):
    f32 = jnp.float32
    ei = edge_index.astype(jnp.int32)
    w = edge_weight.astype(f32)
    rowp, colp, wp, Xf = _prep(ei, w, X[0].reshape(N, 2 * PERIODS))

    col16 = colp.reshape(NS, CH1, 128)
    w16 = wp.reshape(NS, CH1, 128)
    row32 = rowp.reshape(NC * NS, CH3, 128)
    col32 = colp.reshape(NC * NS, CH3, 128)
    w32 = wp.reshape(NC * NS, CH3, 128)

    deg = _k1(col16, w16)
    U, dB = _k2(deg, Xf)
    SP = _k3(row32, col32, w32, U)

    full = lambda s: pl.BlockSpec(s, lambda i: (0,) * len(s))
    out = pl.pallas_call(
        _gru_body,
        grid=(NP // BN,),
        in_specs=[
            pl.BlockSpec((NC, BN, W), lambda i: (0, i, 0)),
            pl.BlockSpec((BN, W), lambda i: (i, 0)),
            pl.BlockSpec((BN, W), lambda i: (i, 0)),
            full((1, PERIODS)),
            full((OUT, 2)), full((1, OUT)), full((OUT, 2 * OUT)), full((1, OUT)),
            full((OUT, 2)), full((1, OUT)), full((OUT, 2 * OUT)), full((1, OUT)),
            full((OUT, 2)), full((1, OUT)), full((OUT, 2 * OUT)), full((1, OUT)),
        ],
        out_specs=pl.BlockSpec((OUT, BN), lambda i: (0, i)),
        out_shape=jax.ShapeDtypeStruct((OUT, N), f32),
    )(SP, U, dB, attention.reshape(1, PERIODS),
      conv_z_W, conv_z_b.reshape(1, OUT), lin_z_W, lin_z_b.reshape(1, OUT),
      conv_r_W, conv_r_b.reshape(1, OUT), lin_r_W, lin_r_b.reshape(1, OUT),
      conv_h_W, conv_h_b.reshape(1, OUT), lin_h_W, lin_h_b.reshape(1, OUT))

    return jnp.transpose(out)[jnp.newaxis]
